# baseline (speedup 1.0000x reference)
import functools

import jax
import jax.numpy as jnp
from jax import lax
from jax.experimental import pallas as pl
from jax.experimental.pallas import tpu as pltpu
from jax.experimental.pallas import tpu_sc as plsc

_B, _S = 4096, 26
_F = 32
_TOTAL = _B * _S
_NC, _NS = 2, 16
_NW = _NC * _NS
_PER_W = _TOTAL // _NW
_CHUNK = 128
_NCHUNK = _PER_W // _CHUNK

_mesh = plsc.VectorSubcoreMesh(core_axis_name="c", subcore_axis_name="s")


@functools.partial(
    pl.kernel,
    out_type=jax.ShapeDtypeStruct((_S, _F, _B), jnp.float32),
    mesh=_mesh,
    scratch_types=[
        pltpu.VMEM((_NCHUNK, _CHUNK), jnp.int32),
        pltpu.VMEM((_PER_W, _F), jnp.float32),
        pltpu.VMEM((2, _F, _CHUNK), jnp.float32),
        pltpu.SemaphoreType.DMA,
        pltpu.SemaphoreType.DMA,
    ],
    compiler_params=pltpu.CompilerParams(
        use_tc_tiling_on_sc=False, needs_layout_passes=False
    ),
)
def _gather_kernel(idx_hbm, table_hbm, out_hbm, idx_v, rows_v, rowsT, gsem, osem):
    wid = lax.axis_index("s") * _NC + lax.axis_index("c")
    pltpu.sync_copy(idx_hbm.at[pl.ds(wid * _NCHUNK, _NCHUNK)], idx_v)
    for j in range(_NCHUNK):
        pltpu.async_copy(
            table_hbm.at[idx_v.at[j]],
            rows_v.at[pl.ds(j * _CHUNK, _CHUNK)],
            gsem,
        )
    lanes = lax.iota(jnp.int32, 16)

    def per_chunk(j, carry):
        pltpu.make_async_copy(
            table_hbm.at[idx_v.at[0]],
            rows_v.at[pl.ds(j * _CHUNK, _CHUNK)],
            gsem,
        ).wait()

        buf = rowsT.at[j & 1]

        @pl.when(j >= 2)
        def _():
            pltpu.make_async_copy(buf, out_hbm.at[0, :, pl.ds(0, _CHUNK)], osem).wait()

        def per_bg(bg, c2):
            base = j * _CHUNK + bg * 16
            row_ids = lanes + base
            for f in range(_F):
                v = plsc.load_gather(
                    rows_v, [row_ids, jnp.full((16,), f, jnp.int32)]
                )
                buf[f, pl.ds(bg * 16, 16)] = v
            return c2

        lax.fori_loop(0, _CHUNK // 16, per_bg, 0)

        u = wid * _NCHUNK + j
        s = u >> 5
        bblk = u & 31
        pltpu.async_copy(
            buf, out_hbm.at[s, :, pl.ds(bblk * _CHUNK, _CHUNK)], osem
        )
        return carry

    lax.fori_loop(0, _NCHUNK, per_chunk, 0)
    for _ in range(2):
        pltpu.make_async_copy(
            rowsT.at[0], out_hbm.at[0, :, pl.ds(0, _CHUNK)], osem
        ).wait()


def kernel(inputs, embedding):
    idx = inputs.T.reshape(_NW * _NCHUNK, _CHUNK)
    out = _gather_kernel(idx, embedding)
    return out.transpose(2, 0, 1)
